# baseline (speedup 1.0000x reference)
import functools

import jax
import jax.numpy as jnp
from jax import lax
from jax.experimental import pallas as pl
from jax.experimental.pallas import tpu as pltpu
from jax.experimental.pallas import tpu_sc as plsc

B, L = 4096, 200
V, D, O = 1000000, 64, 3

NC, NS, LANES = 2, 16, 16
NW = NC * NS
SEG_PER_W = B // NW
HALF = 104
HALF_REAL = L // 2
NROW = 2 * SEG_PER_W
NVREG = D // LANES
NBUF = 4
UNROLL = 4


def _sc_pool(idx_r, table):
    mesh = plsc.VectorSubcoreMesh(core_axis_name="c", subcore_axis_name="s")

    @functools.partial(
        pl.kernel,
        mesh=mesh,
        out_type=jax.ShapeDtypeStruct((B, D), jnp.float32),
        scratch_types=[
            pltpu.VMEM((NROW, HALF), jnp.int32),
            pltpu.VMEM((NBUF, 2, HALF, D), jnp.float32),
            pltpu.VMEM((SEG_PER_W, D), jnp.float32),
            pltpu.SemaphoreType.DMA((NBUF,)),
        ],
        compiler_params=pltpu.CompilerParams(use_tc_tiling_on_sc=False),
    )
    def k(idx_hbm, table_hbm, out_hbm, idx_v, rows, out_v, sems):
        wid = lax.axis_index("s") * NC + lax.axis_index("c")
        pltpu.sync_copy(idx_hbm.at[wid], idx_v)

        def issue(bi, buf):
            pltpu.async_copy(
                table_hbm.at[idx_v.at[2 * bi]], rows.at[buf, 0], sems.at[buf])
            pltpu.async_copy(
                table_hbm.at[idx_v.at[2 * bi + 1]], rows.at[buf, 1], sems.at[buf])

        def drain(buf):
            pltpu.make_async_copy(
                table_hbm.at[idx_v.at[0]], rows.at[buf, 0], sems.at[buf]).wait()
            pltpu.make_async_copy(
                table_hbm.at[idx_v.at[0]], rows.at[buf, 1], sems.at[buf]).wait()

        def accum_store(bi, buf):
            r0 = rows.at[buf, 0]
            r1 = rows.at[buf, 1]

            def inner(j, acc):
                new = []
                for kk in range(NVREG):
                    a = acc[kk]
                    for u in range(UNROLL):
                        a = (a
                             + r0[j * UNROLL + u, pl.ds(kk * LANES, LANES)]
                             + r1[j * UNROLL + u, pl.ds(kk * LANES, LANES)])
                    new.append(a)
                return tuple(new)

            zero = jnp.zeros((LANES,), jnp.float32)
            acc = lax.fori_loop(0, HALF_REAL // UNROLL, inner, (zero,) * NVREG)
            for kk in range(NVREG):
                out_v[bi, pl.ds(kk * LANES, LANES)] = acc[kk]

        for buf in range(NBUF):
            issue(buf, buf)

        def outer(it, carry):
            bi0 = it * NBUF
            for u in range(NBUF):
                bi = bi0 + u
                drain(u)
                accum_store(bi, u)
                nxt = bi + NBUF

                @pl.when(nxt < SEG_PER_W)
                def _():
                    issue(nxt, u)
            return carry

        lax.fori_loop(0, SEG_PER_W // NBUF, outer, 0)
        pltpu.sync_copy(out_v, out_hbm.at[pl.ds(wid * SEG_PER_W, SEG_PER_W)])

    return k(idx_r, table)


def _tc_linear_body(pooled_ref, mask_ref, w_ref, b_ref, out_ref):
    lens = jnp.sum(mask_ref[...], axis=1, keepdims=True)
    pooled = pooled_ref[...] / lens
    out_ref[...] = (
        jnp.dot(pooled, w_ref[...], preferred_element_type=jnp.float32)
        + b_ref[...]
    )


def _tc_linear(pooled, text_mask, W, b):
    return pl.pallas_call(
        _tc_linear_body,
        out_shape=jax.ShapeDtypeStruct((B, O), jnp.float32),
    )(pooled, text_mask, W, b.reshape(1, O))


def kernel(topic, topic_mask, text, text_mask, embedding, W, b):
    idx = text.astype(jnp.int32).reshape(B, 2, HALF_REAL)
    idx = jnp.pad(idx, ((0, 0), (0, 0), (0, HALF - HALF_REAL)))
    idx = idx.reshape(NW, NROW, HALF)
    pooled = _sc_pool(idx, embedding)
    return _tc_linear(pooled, text_mask.astype(jnp.float32), W, b)


# speedup vs baseline: 1.1305x; 1.1305x over previous
import functools

import jax
import jax.numpy as jnp
from jax import lax
from jax.experimental import pallas as pl
from jax.experimental.pallas import tpu as pltpu
from jax.experimental.pallas import tpu_sc as plsc

B, L = 4096, 200
V, D, O = 1000000, 64, 3
OP = 16

NC, NS, LANES = 2, 16, 16
NW = NC * NS
SEG_PER_W = B // NW
HALF = 104
HALF_REAL = L // 2
NROW = 2 * SEG_PER_W
NBUF = 4
CHUNKS = HALF_REAL * OP // LANES

MM_BLOCK = 10000


def _ew_body(x_ref, w_ref, o_ref):
    o_ref[...] = jnp.dot(x_ref[...], w_ref[...],
                         preferred_element_type=jnp.float32)


def _ew_table(embedding, Wp):
    return pl.pallas_call(
        _ew_body,
        grid=(V // MM_BLOCK,),
        in_specs=[
            pl.BlockSpec((MM_BLOCK, D), lambda i: (i, 0)),
            pl.BlockSpec((D, OP), lambda i: (0, 0)),
        ],
        out_specs=pl.BlockSpec((MM_BLOCK, OP), lambda i: (i, 0)),
        out_shape=jax.ShapeDtypeStruct((V, OP), jnp.float32),
    )(embedding, Wp)


def _sc_pool(idx_r, ew):
    mesh = plsc.VectorSubcoreMesh(core_axis_name="c", subcore_axis_name="s")

    @functools.partial(
        pl.kernel,
        mesh=mesh,
        out_type=jax.ShapeDtypeStruct((B, LANES), jnp.float32),
        scratch_types=[
            pltpu.VMEM((NROW, HALF), jnp.int32),
            pltpu.VMEM((NBUF * 2 * HALF, OP), jnp.float32),
            pltpu.VMEM((SEG_PER_W, LANES), jnp.float32),
            pltpu.SemaphoreType.DMA((NBUF,)),
        ],
        compiler_params=pltpu.CompilerParams(
            use_tc_tiling_on_sc=False, needs_layout_passes=False),
    )
    def k(idx_hbm, ew_hbm, out_hbm, idx_v, rows, out_v, sems):
        wid = lax.axis_index("s") * NC + lax.axis_index("c")
        pltpu.sync_copy(idx_hbm.at[wid], idx_v)

        def base(buf, h):
            return (buf * 2 + h) * HALF

        def issue(bi, buf):
            pltpu.async_copy(
                ew_hbm.at[idx_v.at[2 * bi]],
                rows.at[pl.ds(base(buf, 0), HALF)], sems.at[buf])
            pltpu.async_copy(
                ew_hbm.at[idx_v.at[2 * bi + 1]],
                rows.at[pl.ds(base(buf, 1), HALF)], sems.at[buf])

        def drain(buf):
            pltpu.make_async_copy(
                ew_hbm.at[idx_v.at[0]],
                rows.at[pl.ds(base(buf, 0), HALF)], sems.at[buf]).wait()
            pltpu.make_async_copy(
                ew_hbm.at[idx_v.at[0]],
                rows.at[pl.ds(base(buf, 1), HALF)], sems.at[buf]).wait()

        def accum_store(bi, buf):
            acc = jnp.zeros((LANES,), jnp.float32)
            for h in range(2):
                for j in range(HALF_REAL):
                    acc = acc + rows[base(buf, h) + j, :]
            out_v[bi, :] = acc

        for buf in range(NBUF):
            issue(buf, buf)

        def outer(it, carry):
            bi0 = it * NBUF
            for u in range(NBUF):
                bi = bi0 + u
                drain(u)
                accum_store(bi, u)
                nxt = bi + NBUF

                @pl.when(nxt < SEG_PER_W)
                def _():
                    issue(nxt, u)
            return carry

        lax.fori_loop(0, SEG_PER_W // NBUF, outer, 0)
        pltpu.sync_copy(out_v, out_hbm.at[pl.ds(wid * SEG_PER_W, SEG_PER_W)])

    return k(idx_r, ew)


def _finish_body(p_ref, mask_ref, b_ref, o_ref):
    lens = jnp.sum(mask_ref[...], axis=1, keepdims=True)
    p = p_ref[...]
    o_ref[...] = p[:, :O] / lens + b_ref[...]


def _finish(pooled16, text_mask, b):
    return pl.pallas_call(
        _finish_body,
        out_shape=jax.ShapeDtypeStruct((B, O), jnp.float32),
    )(pooled16, text_mask, b.reshape(1, O))


def kernel(topic, topic_mask, text, text_mask, embedding, W, b):
    Wp = jnp.pad(W.astype(jnp.float32), ((0, 0), (0, OP - O)))
    ew = _ew_table(embedding, Wp)
    idx = text.astype(jnp.int32).reshape(B, 2, HALF_REAL)
    idx = jnp.pad(idx, ((0, 0), (0, 0), (0, HALF - HALF_REAL)))
    idx = idx.reshape(NW, NROW, HALF)
    pooled16 = _sc_pool(idx, ew)
    return _finish(pooled16, text_mask.astype(jnp.float32), b)


# speedup vs baseline: 1.6630x; 1.4710x over previous
import functools

import jax
import jax.numpy as jnp
from jax import lax
from jax.experimental import pallas as pl
from jax.experimental.pallas import tpu as pltpu
from jax.experimental.pallas import tpu_sc as plsc

B, L = 4096, 200
V, D, O = 1000000, 64, 3
OP = 16

NC, NS, LANES = 2, 16, 16
NW = NC * NS
SEG_PER_W = B // NW
HALF = 104
HALF_REAL = L // 2
NROW = 2 * SEG_PER_W
NBUF = 4
CHUNKS = HALF_REAL * OP // LANES

MM_BLOCK = 8192


def _ew_body(xt_ref, w_ref, o_ref):
    o_ref[...] = lax.dot_general(
        xt_ref[...], w_ref[...], (((0,), (0,)), ((), ())),
        preferred_element_type=jnp.float32)


def _ew_table(embeddingT, Wp):
    return pl.pallas_call(
        _ew_body,
        grid=(pl.cdiv(V, MM_BLOCK),),
        in_specs=[
            pl.BlockSpec((D, MM_BLOCK), lambda i: (0, i)),
            pl.BlockSpec((D, OP), lambda i: (0, 0)),
        ],
        out_specs=pl.BlockSpec((MM_BLOCK, OP), lambda i: (i, 0)),
        out_shape=jax.ShapeDtypeStruct((V, OP), jnp.float32),
    )(embeddingT, Wp)


def _sc_pool(idx_r, ew):
    mesh = plsc.VectorSubcoreMesh(core_axis_name="c", subcore_axis_name="s")

    @functools.partial(
        pl.kernel,
        mesh=mesh,
        out_type=jax.ShapeDtypeStruct((B, LANES), jnp.float32),
        scratch_types=[
            pltpu.VMEM((NROW, HALF), jnp.int32),
            pltpu.VMEM((NBUF * 2 * HALF, OP), jnp.float32),
            pltpu.VMEM((SEG_PER_W, LANES), jnp.float32),
            pltpu.SemaphoreType.DMA((NBUF,)),
        ],
        compiler_params=pltpu.CompilerParams(
            use_tc_tiling_on_sc=False, needs_layout_passes=False),
    )
    def k(idx_hbm, ew_hbm, out_hbm, idx_v, rows, out_v, sems):
        wid = lax.axis_index("s") * NC + lax.axis_index("c")
        pltpu.sync_copy(idx_hbm.at[wid], idx_v)

        def base(buf, h):
            return (buf * 2 + h) * HALF

        def issue(bi, buf):
            pltpu.async_copy(
                ew_hbm.at[idx_v.at[2 * bi]],
                rows.at[pl.ds(base(buf, 0), HALF)], sems.at[buf])
            pltpu.async_copy(
                ew_hbm.at[idx_v.at[2 * bi + 1]],
                rows.at[pl.ds(base(buf, 1), HALF)], sems.at[buf])

        def drain(buf):
            pltpu.make_async_copy(
                ew_hbm.at[idx_v.at[0]],
                rows.at[pl.ds(base(buf, 0), HALF)], sems.at[buf]).wait()
            pltpu.make_async_copy(
                ew_hbm.at[idx_v.at[0]],
                rows.at[pl.ds(base(buf, 1), HALF)], sems.at[buf]).wait()

        def accum_store(bi, buf):
            acc = jnp.zeros((LANES,), jnp.float32)
            for h in range(2):
                for j in range(HALF_REAL):
                    acc = acc + rows[base(buf, h) + j, :]
            out_v[bi, :] = acc

        for buf in range(NBUF):
            issue(buf, buf)

        def outer(it, carry):
            bi0 = it * NBUF
            for u in range(NBUF):
                bi = bi0 + u
                drain(u)
                accum_store(bi, u)
                nxt = bi + NBUF

                @pl.when(nxt < SEG_PER_W)
                def _():
                    issue(nxt, u)
            return carry

        lax.fori_loop(0, SEG_PER_W // NBUF, outer, 0)
        pltpu.sync_copy(out_v, out_hbm.at[pl.ds(wid * SEG_PER_W, SEG_PER_W)])

    return k(idx_r, ew)


def _finish_body(p_ref, mask_ref, b_ref, o_ref):
    lens = jnp.sum(mask_ref[...], axis=1, keepdims=True)
    p = p_ref[...]
    o_ref[...] = p[:, :O] / lens + b_ref[...]


def _finish(pooled16, text_mask, b):
    return pl.pallas_call(
        _finish_body,
        out_shape=jax.ShapeDtypeStruct((B, O), jnp.float32),
    )(pooled16, text_mask, b.reshape(1, O))


def kernel(topic, topic_mask, text, text_mask, embedding, W, b):
    Wp = jnp.pad(W.astype(jnp.float32), ((0, 0), (0, OP - O)))
    ew = _ew_table(embedding.T, Wp)
    idx = text.astype(jnp.int32).reshape(B, 2, HALF_REAL)
    idx = jnp.pad(idx, ((0, 0), (0, 0), (0, HALF - HALF_REAL)))
    idx = idx.reshape(NW, NROW, HALF)
    pooled16 = _sc_pool(idx, ew)
    return _finish(pooled16, text_mask.astype(jnp.float32), b)
